# baseline (speedup 1.0000x reference)
import dataclasses
import functools

import jax
import jax.numpy as jnp
import numpy as np
from jax import lax
from jax.experimental import pallas as pl
from jax.experimental.pallas import tpu as pltpu
from jax.experimental.pallas import tpu_sc as plsc

_LEVELS = 1024
_D = 8192
_T = 2048
_FEAT_SEL = np.array([558, 582, 554, 552, 93, 555, 580, 571, 574, 578,
                      566, 287, 556, 550, 14, 551, 64, 581])

_NTILES = 32
_TPT = _T // _NTILES
_RPT = _TPT * 3
_NSPLIT = 8
_W = _D // _NSPLIT
_G = 8
_NGROUP = _TPT // _G
_TC_CHUNK = 128
_TC_STEPS = _T // _TC_CHUNK


def _quant_body(x_ref, o_ref):
    o_ref[...] = jnp.clip(
        jnp.floor(x_ref[...] * float(_LEVELS)).astype(jnp.int32), 0, _LEVELS - 1)


def _quantize(vals):
    return pl.pallas_call(
        _quant_body,
        out_shape=jax.ShapeDtypeStruct(vals.shape, jnp.int32),
    )(vals)


def _sc_bind_gather(level2, ftable2, keys_hv, gmain, gfeat):
    mesh = plsc.VectorSubcoreMesh(core_axis_name="c", subcore_axis_name="s")

    @functools.partial(
        pl.kernel,
        mesh=mesh,
        out_type=[
            jax.ShapeDtypeStruct((_T, _D), jnp.float32),
            jax.ShapeDtypeStruct((24, _D), jnp.float32),
        ],
        scratch_types=[
            pltpu.VMEM((208,), jnp.int32),
            pltpu.VMEM((_NSPLIT, _NGROUP, 32), jnp.int32),
            pltpu.VMEM((24, 8), jnp.int32),
            pltpu.VMEM((24, _W), jnp.float32),
            pltpu.VMEM((24, _W), jnp.float32),
            pltpu.VMEM((1, _D), jnp.float32),
            pltpu.VMEM((_G, _W), jnp.float32),
            pltpu.VMEM((_G, _W), jnp.float32),
            pltpu.VMEM((3, _D), jnp.float32),
            pltpu.SemaphoreType.DMA,
            pltpu.SemaphoreType.DMA,
            pltpu.SemaphoreType.DMA,
            pltpu.SemaphoreType.DMA,
        ],
    )
    def gather_k(table_hbm, ftable_hbm, keys_hbm, gmain_hbm, gfeat_hbm,
                 hv_out, feat_out, idx_v, idxq_v, fidx_v,
                 rows0_v, rows1_v, frow_v, h0_v, h1_v, keys_v,
                 sem0, sem1, osem0, osem1):
        cid = lax.axis_index("c")
        sid = lax.axis_index("s")
        wid = sid * 2 + cid
        t0 = wid * _TPT
        base = wid * _RPT
        bufs = (rows0_v, rows1_v)
        sems = (sem0, sem1)
        hbufs = (h0_v, h1_v)
        osems = (osem0, osem1)
        pltpu.sync_copy(keys_hbm, keys_v)
        pltpu.sync_copy(gmain_hbm.at[pl.ds(base, _RPT)], idx_v.at[pl.ds(0, _RPT)])
        for q in range(_NSPLIT):
            for g in range(_NGROUP):
                for k in range(2):
                    j = idx_v[pl.ds(24 * g + 16 * k, 16)]
                    idxq_v[q, g, pl.ds(16 * k, 16)] = j * _NSPLIT + q

        @pl.when(wid < 8)
        def _():
            pltpu.sync_copy(gfeat_hbm, fidx_v)
            for j in range(3):
                row = wid * 3 + j
                pltpu.async_copy(
                    ftable_hbm.at[fidx_v.at[row, pl.ds(0, 1)]],
                    frow_v, sem0).wait()
                pltpu.sync_copy(frow_v, feat_out.at[pl.ds(row, 1)])

        pltpu.async_copy(table_hbm.at[idxq_v.at[0, 0, pl.ds(0, 24)]],
                         rows0_v, sem0)

        @pl.loop(0, _NGROUP)
        def _(g):
            for q in range(_NSPLIT):
                cur = q % 2
                nxt = (q + 1) % 2
                pltpu.make_async_copy(
                    table_hbm.at[idxq_v.at[q, g, pl.ds(0, 24)]],
                    bufs[cur], sems[cur]).wait()
                if q < _NSPLIT - 1:
                    pltpu.async_copy(
                        table_hbm.at[idxq_v.at[q + 1, g, pl.ds(0, 24)]],
                        bufs[nxt], sems[nxt])
                else:
                    @pl.when(g < _NGROUP - 1)
                    def _():
                        pltpu.async_copy(
                            table_hbm.at[idxq_v.at[0, g + 1, pl.ds(0, 24)]],
                            bufs[nxt], sems[nxt])

                rows_v = bufs[cur]
                h_v = hbufs[cur]
                osem = osems[cur]
                if q >= 2:
                    pltpu.make_async_copy(
                        h_v,
                        hv_out.at[pl.ds(t0 + g * _G, _G),
                                  pl.ds((q - 2) * _W, _W)],
                        osem).wait()
                else:
                    @pl.when(g > 0)
                    def _():
                        pltpu.make_async_copy(
                            h_v,
                            hv_out.at[pl.ds(t0, _G), pl.ds(q * _W, _W)],
                            osem).wait()

                @pl.loop(0, _W // 32)
                def _(dc2):
                    for u in range(2):
                        off = dc2 * 32 + u * 16
                        koff = q * _W + off
                        k0 = keys_v[0, pl.ds(koff, 16)]
                        k1 = keys_v[1, pl.ds(koff, 16)]
                        k2 = keys_v[2, pl.ds(koff, 16)]
                        for r in range(_G):
                            h_v[r, pl.ds(off, 16)] = (
                                k0 * rows_v[3 * r, pl.ds(off, 16)]
                                + k1 * rows_v[3 * r + 1, pl.ds(off, 16)]
                                + k2 * rows_v[3 * r + 2, pl.ds(off, 16)])

                pltpu.async_copy(
                    h_v,
                    hv_out.at[pl.ds(t0 + g * _G, _G), pl.ds(q * _W, _W)],
                    osem)

        pltpu.make_async_copy(
            h0_v, hv_out.at[pl.ds(t0, _G), pl.ds(0, _W)], osem0).wait()
        pltpu.make_async_copy(
            h1_v, hv_out.at[pl.ds(t0, _G), pl.ds(_W, _W)], osem1).wait()

    return gather_k(level2, ftable2, keys_hv, gmain, gfeat)


def _dense_body(h_ref, f_ref, fk_ref, o_ref, carry_ref, acc_ref):
    s = pl.program_id(0)

    @pl.when(s == 0)
    def _():
        carry_ref[...] = jnp.zeros_like(carry_ref)
        acc_ref[...] = jnp.zeros_like(acc_ref)

    h = h_ref[...]
    hcat = jnp.concatenate([carry_ref[...], h], axis=0)
    u = hcat[0:_TC_CHUNK]
    v = hcat[1:_TC_CHUNK + 1]
    w = hcat[2:_TC_CHUNK + 2]
    ur = jnp.concatenate([u[:, -2:], u[:, :-2]], axis=1)
    vr = jnp.concatenate([v[:, -1:], v[:, :-1]], axis=1)
    term = ur * vr * w
    part = acc_ref[...]
    for c in range(_TC_CHUNK // 8):
        part = part + term[c * 8:(c + 1) * 8]
    acc_ref[...] = part
    carry_ref[...] = h[_TC_CHUNK - 2:_TC_CHUNK]

    @pl.when(s == _TC_STEPS - 1)
    def _():
        fhv = jnp.sum(f_ref[...] * fk_ref[...], axis=0, keepdims=True)
        shv = jnp.sum(acc_ref[...], axis=0, keepdims=True)
        comb = shv + fhv + shv * fhv
        o_ref[...] = jnp.where(comb > 0, 1.0, -1.0)


def _dense(hv, feat_rows, feat_keys_pad):
    return pl.pallas_call(
        _dense_body,
        grid=(_TC_STEPS,),
        in_specs=[
            pl.BlockSpec((_TC_CHUNK, _D), lambda s: (s, 0)),
            pl.BlockSpec((24, _D), lambda s: (0, 0)),
            pl.BlockSpec((24, _D), lambda s: (0, 0)),
        ],
        out_specs=pl.BlockSpec((1, _D), lambda s: (0, 0)),
        out_shape=jax.ShapeDtypeStruct((1, _D), jnp.float32),
        scratch_shapes=[
            pltpu.VMEM((2, _D), jnp.float32),
            pltpu.VMEM((8, _D), jnp.float32),
        ],
    )(hv, feat_rows, feat_keys_pad)


@jax.jit
def _run(signals, feat, keys_hv, level_hvs, feat_keys, feat_level_hvs):
    f18 = feat[_FEAT_SEL]
    fpad = jnp.concatenate([f18, jnp.zeros((14,), jnp.float32)]).reshape(8, 4)
    x = jnp.concatenate([signals, fpad], axis=0)
    idx = _quantize(x)
    gmain = idx[:_T, 1:4].reshape(_T * 3)
    gfeat = jnp.zeros((24, 8), jnp.int32).at[:, 0].set(
        idx[_T:_T + 6].reshape(24))
    level2 = level_hvs.reshape(_LEVELS * _NSPLIT, _W)
    hv, feat_rows = _sc_bind_gather(level2, feat_level_hvs, keys_hv,
                                    gmain, gfeat)
    fk_pad = jnp.concatenate(
        [feat_keys, jnp.zeros((6, _D), jnp.float32)], axis=0)
    out = _dense(hv, feat_rows, fk_pad)
    return out.reshape(_D)


def kernel(signals, feat, keys_hv, level_hvs, feat_keys, feat_level_hvs):
    return _run(signals, feat, keys_hv, level_hvs, feat_keys, feat_level_hvs)


# speedup vs baseline: 1.4361x; 1.4361x over previous
import dataclasses
import functools

import jax
import jax.numpy as jnp
import numpy as np
from jax import lax
from jax.experimental import pallas as pl
from jax.experimental.pallas import tpu as pltpu
from jax.experimental.pallas import tpu_sc as plsc

_LEVELS = 1024
_D = 8192
_T = 2048
_FEAT_SEL = np.array([558, 582, 554, 552, 93, 555, 580, 571, 574, 578,
                      566, 287, 556, 550, 14, 551, 64, 581])

_NTILES = 32
_TPT = _T // _NTILES
_RPT = _TPT * 3
_NSPLIT = 8
_W = _D // _NSPLIT
_G = 8
_NGROUP = _TPT // _G
_TC_CHUNK = 128
_TC_STEPS = _T // _TC_CHUNK


def _quant_body(x_ref, o_ref):
    o_ref[...] = jnp.clip(
        jnp.floor(x_ref[...] * float(_LEVELS)).astype(jnp.int32), 0, _LEVELS - 1)


def _quantize(vals):
    return pl.pallas_call(
        _quant_body,
        out_shape=jax.ShapeDtypeStruct(vals.shape, jnp.int32),
    )(vals)


def _sc_bind_gather(level2, ftable2, keys_hv, gmain, gfeat):
    mesh = plsc.VectorSubcoreMesh(core_axis_name="c", subcore_axis_name="s")

    @functools.partial(
        pl.kernel,
        mesh=mesh,
        out_type=[
            jax.ShapeDtypeStruct((_T, _D), jnp.float32),
            jax.ShapeDtypeStruct((24, _D), jnp.float32),
        ],
        scratch_types=[
            pltpu.VMEM((208,), jnp.int32),
            pltpu.VMEM((_NSPLIT, _NGROUP, 32), jnp.int32),
            pltpu.VMEM((24, 8), jnp.int32),
            pltpu.VMEM((24, _W), jnp.float32),
            pltpu.VMEM((24, _W), jnp.float32),
            pltpu.VMEM((1, _D), jnp.float32),
            pltpu.VMEM((_G, _W), jnp.float32),
            pltpu.VMEM((_G, _W), jnp.float32),
            pltpu.VMEM((3, _D), jnp.float32),
            pltpu.SemaphoreType.DMA,
            pltpu.SemaphoreType.DMA,
            pltpu.SemaphoreType.DMA,
            pltpu.SemaphoreType.DMA,
        ],
    )
    def gather_k(table_hbm, ftable_hbm, keys_hbm, gmain_hbm, gfeat_hbm,
                 hv_out, feat_out, idx_v, idxq_v, fidx_v,
                 rows0_v, rows1_v, frow_v, h0_v, h1_v, keys_v,
                 sem0, sem1, osem0, osem1):
        cid = lax.axis_index("c")
        sid = lax.axis_index("s")
        wid = sid * 2 + cid
        t0 = wid * _TPT
        base = wid * _RPT
        bufs = (rows0_v, rows1_v)
        sems = (sem0, sem1)
        hbufs = (h0_v, h1_v)
        osems = (osem0, osem1)
        pltpu.sync_copy(keys_hbm, keys_v)
        pltpu.sync_copy(gmain_hbm.at[pl.ds(base, _RPT)], idx_v.at[pl.ds(0, _RPT)])
        for q in range(_NSPLIT):
            for g in range(_NGROUP):
                for k in range(2):
                    j = idx_v[pl.ds(24 * g + 16 * k, 16)]
                    idxq_v[q, g, pl.ds(16 * k, 16)] = j * _NSPLIT + q

        @pl.when(wid < 8)
        def _():
            pltpu.sync_copy(gfeat_hbm, fidx_v)
            for j in range(3):
                row = wid * 3 + j
                pltpu.async_copy(
                    ftable_hbm.at[fidx_v.at[row, pl.ds(0, 1)]],
                    frow_v, sem0).wait()
                pltpu.sync_copy(frow_v, feat_out.at[pl.ds(row, 1)])

        pltpu.async_copy(table_hbm.at[idxq_v.at[0, 0, pl.ds(0, 24)]],
                         rows0_v, sem0)

        @pl.loop(0, _NGROUP)
        def _(g):
            for q in range(_NSPLIT):
                cur = q % 2
                nxt = (q + 1) % 2
                pltpu.make_async_copy(
                    table_hbm.at[idxq_v.at[q, g, pl.ds(0, 24)]],
                    bufs[cur], sems[cur]).wait()
                if q < _NSPLIT - 1:
                    pltpu.async_copy(
                        table_hbm.at[idxq_v.at[q + 1, g, pl.ds(0, 24)]],
                        bufs[nxt], sems[nxt])
                else:
                    @pl.when(g < _NGROUP - 1)
                    def _():
                        pltpu.async_copy(
                            table_hbm.at[idxq_v.at[0, g + 1, pl.ds(0, 24)]],
                            bufs[nxt], sems[nxt])

                rows_v = bufs[cur]
                h_v = hbufs[cur]
                osem = osems[cur]
                if q >= 2:
                    pltpu.make_async_copy(
                        h_v,
                        hv_out.at[pl.ds(t0 + g * _G, _G),
                                  pl.ds((q - 2) * _W, _W)],
                        osem).wait()
                else:
                    @pl.when(g > 0)
                    def _():
                        pltpu.make_async_copy(
                            h_v,
                            hv_out.at[pl.ds(t0, _G), pl.ds(q * _W, _W)],
                            osem).wait()

                @pl.loop(0, _W // 16)
                def _(dc):
                    koff = q * _W + dc * 16
                    k0 = keys_v[0, pl.ds(koff, 16)]
                    k1 = keys_v[1, pl.ds(koff, 16)]
                    k2 = keys_v[2, pl.ds(koff, 16)]
                    for r in range(_G):
                        h_v[r, pl.ds(dc * 16, 16)] = (
                            k0 * rows_v[3 * r, pl.ds(dc * 16, 16)]
                            + k1 * rows_v[3 * r + 1, pl.ds(dc * 16, 16)]
                            + k2 * rows_v[3 * r + 2, pl.ds(dc * 16, 16)])

                pltpu.async_copy(
                    h_v,
                    hv_out.at[pl.ds(t0 + g * _G, _G), pl.ds(q * _W, _W)],
                    osem)

        pltpu.make_async_copy(
            h0_v, hv_out.at[pl.ds(t0, _G), pl.ds(0, _W)], osem0).wait()
        pltpu.make_async_copy(
            h1_v, hv_out.at[pl.ds(t0, _G), pl.ds(_W, _W)], osem1).wait()

    return gather_k(level2, ftable2, keys_hv, gmain, gfeat)


def _dense_body(h_ref, f_ref, fk_ref, o_ref, carry_ref, acc_ref):
    s = pl.program_id(0)

    @pl.when(s == 0)
    def _():
        carry_ref[...] = jnp.zeros_like(carry_ref)
        acc_ref[...] = jnp.zeros_like(acc_ref)

    h = h_ref[...]
    hcat = jnp.concatenate([carry_ref[...], h], axis=0)
    u = hcat[0:_TC_CHUNK]
    v = hcat[1:_TC_CHUNK + 1]
    w = hcat[2:_TC_CHUNK + 2]
    ur = jnp.concatenate([u[:, -2:], u[:, :-2]], axis=1)
    vr = jnp.concatenate([v[:, -1:], v[:, :-1]], axis=1)
    term = ur * vr * w
    part = acc_ref[...]
    for c in range(_TC_CHUNK // 8):
        part = part + term[c * 8:(c + 1) * 8]
    acc_ref[...] = part
    carry_ref[...] = h[_TC_CHUNK - 2:_TC_CHUNK]

    @pl.when(s == _TC_STEPS - 1)
    def _():
        fhv = jnp.sum(f_ref[...] * fk_ref[...], axis=0, keepdims=True)
        shv = jnp.sum(acc_ref[...], axis=0, keepdims=True)
        comb = shv + fhv + shv * fhv
        o_ref[...] = jnp.where(comb > 0, 1.0, -1.0)


def _dense(hv, feat_rows, feat_keys_pad):
    return pl.pallas_call(
        _dense_body,
        grid=(_TC_STEPS,),
        in_specs=[
            pl.BlockSpec((_TC_CHUNK, _D), lambda s: (s, 0)),
            pl.BlockSpec((24, _D), lambda s: (0, 0)),
            pl.BlockSpec((24, _D), lambda s: (0, 0)),
        ],
        out_specs=pl.BlockSpec((1, _D), lambda s: (0, 0)),
        out_shape=jax.ShapeDtypeStruct((1, _D), jnp.float32),
        scratch_shapes=[
            pltpu.VMEM((2, _D), jnp.float32),
            pltpu.VMEM((8, _D), jnp.float32),
        ],
    )(hv, feat_rows, feat_keys_pad)


@jax.jit
def _run(signals, feat, keys_hv, level_hvs, feat_keys, feat_level_hvs):
    f18 = feat[_FEAT_SEL]
    fpad = jnp.concatenate([f18, jnp.zeros((14,), jnp.float32)]).reshape(8, 4)
    x = jnp.concatenate([signals, fpad], axis=0)
    idx = _quantize(x)
    gmain = idx[:_T, 1:4].reshape(_T * 3)
    gfeat = jnp.zeros((24, 8), jnp.int32).at[:, 0].set(
        idx[_T:_T + 6].reshape(24))
    level2 = level_hvs.reshape(_LEVELS * _NSPLIT, _W)
    hv, feat_rows = _sc_bind_gather(level2, feat_level_hvs, keys_hv,
                                    gmain, gfeat)
    fk_pad = jnp.concatenate(
        [feat_keys, jnp.zeros((6, _D), jnp.float32)], axis=0)
    out = _dense(hv, feat_rows, fk_pad)
    return out.reshape(_D)


def kernel(signals, feat, keys_hv, level_hvs, feat_keys, feat_level_hvs):
    return _run(signals, feat, keys_hv, level_hvs, feat_keys, feat_level_hvs)


# speedup vs baseline: 1.5404x; 1.0726x over previous
import dataclasses
import functools

import jax
import jax.numpy as jnp
import numpy as np
from jax import lax
from jax.experimental import pallas as pl
from jax.experimental.pallas import tpu as pltpu
from jax.experimental.pallas import tpu_sc as plsc

_LEVELS = 1024
_D = 8192
_T = 2048
_FEAT_SEL = np.array([558, 582, 554, 552, 93, 555, 580, 571, 574, 578,
                      566, 287, 556, 550, 14, 551, 64, 581])

_NTILES = 32
_TPT = _T // _NTILES
_RPT = _TPT * 3
_NSPLIT = 8
_W = _D // _NSPLIT
_G = 8
_NGROUP = _TPT // _G
_TC_CHUNK = 128
_TC_STEPS = _T // _TC_CHUNK


def _quant_body(x_ref, o_ref):
    o_ref[...] = jnp.clip(
        jnp.floor(x_ref[...] * float(_LEVELS)).astype(jnp.int32), 0, _LEVELS - 1)


def _quantize(vals):
    return pl.pallas_call(
        _quant_body,
        out_shape=jax.ShapeDtypeStruct(vals.shape, jnp.int32),
    )(vals)


def _retile_body(x_ref, o_ref):
    o_ref[...] = x_ref[...].reshape(_LEVELS, _W)


def _retile(level_hvs):
    return pl.pallas_call(
        _retile_body,
        grid=(_NSPLIT,),
        in_specs=[pl.BlockSpec((_LEVELS // _NSPLIT, _D), lambda s: (s, 0))],
        out_specs=pl.BlockSpec((_LEVELS, _W), lambda s: (s, 0)),
        out_shape=jax.ShapeDtypeStruct((_LEVELS * _NSPLIT, _W), jnp.float32),
    )(level_hvs)


def _sc_bind_gather(level2, ftable2, keys_hv, gmain, gfeat):
    mesh = plsc.VectorSubcoreMesh(core_axis_name="c", subcore_axis_name="s")

    @functools.partial(
        pl.kernel,
        mesh=mesh,
        out_type=[
            jax.ShapeDtypeStruct((_T, _D), jnp.float32),
            jax.ShapeDtypeStruct((24, _D), jnp.float32),
        ],
        scratch_types=[
            pltpu.VMEM((208,), jnp.int32),
            pltpu.VMEM((_NSPLIT, _NGROUP, 32), jnp.int32),
            pltpu.VMEM((24, 8), jnp.int32),
            pltpu.VMEM((24, _W), jnp.float32),
            pltpu.VMEM((24, _W), jnp.float32),
            pltpu.VMEM((1, _D), jnp.float32),
            pltpu.VMEM((_G, _W), jnp.float32),
            pltpu.VMEM((_G, _W), jnp.float32),
            pltpu.VMEM((3, _D), jnp.float32),
            pltpu.SemaphoreType.DMA,
            pltpu.SemaphoreType.DMA,
            pltpu.SemaphoreType.DMA,
            pltpu.SemaphoreType.DMA,
        ],
    )
    def gather_k(table_hbm, ftable_hbm, keys_hbm, gmain_hbm, gfeat_hbm,
                 hv_out, feat_out, idx_v, idxq_v, fidx_v,
                 rows0_v, rows1_v, frow_v, h0_v, h1_v, keys_v,
                 sem0, sem1, osem0, osem1):
        cid = lax.axis_index("c")
        sid = lax.axis_index("s")
        wid = sid * 2 + cid
        t0 = wid * _TPT
        base = wid * _RPT
        bufs = (rows0_v, rows1_v)
        sems = (sem0, sem1)
        hbufs = (h0_v, h1_v)
        osems = (osem0, osem1)
        pltpu.sync_copy(keys_hbm, keys_v)
        pltpu.sync_copy(gmain_hbm.at[pl.ds(base, _RPT)], idx_v.at[pl.ds(0, _RPT)])
        for q in range(_NSPLIT):
            for g in range(_NGROUP):
                for k in range(2):
                    j = idx_v[pl.ds(24 * g + 16 * k, 16)]
                    idxq_v[q, g, pl.ds(16 * k, 16)] = j * _NSPLIT + q

        @pl.when(wid < 8)
        def _():
            pltpu.sync_copy(gfeat_hbm, fidx_v)
            for j in range(3):
                row = wid * 3 + j
                pltpu.async_copy(
                    ftable_hbm.at[fidx_v.at[row, pl.ds(0, 1)]],
                    frow_v, sem0).wait()
                pltpu.sync_copy(frow_v, feat_out.at[pl.ds(row, 1)])

        pltpu.async_copy(table_hbm.at[idxq_v.at[0, 0, pl.ds(0, 24)]],
                         rows0_v, sem0)

        @pl.loop(0, _NGROUP)
        def _(g):
            for q in range(_NSPLIT):
                cur = q % 2
                nxt = (q + 1) % 2
                pltpu.make_async_copy(
                    table_hbm.at[idxq_v.at[q, g, pl.ds(0, 24)]],
                    bufs[cur], sems[cur]).wait()
                if q < _NSPLIT - 1:
                    pltpu.async_copy(
                        table_hbm.at[idxq_v.at[q + 1, g, pl.ds(0, 24)]],
                        bufs[nxt], sems[nxt])
                else:
                    @pl.when(g < _NGROUP - 1)
                    def _():
                        pltpu.async_copy(
                            table_hbm.at[idxq_v.at[0, g + 1, pl.ds(0, 24)]],
                            bufs[nxt], sems[nxt])

                rows_v = bufs[cur]
                h_v = hbufs[cur]
                osem = osems[cur]
                if q >= 2:
                    pltpu.make_async_copy(
                        h_v,
                        hv_out.at[pl.ds(t0 + g * _G, _G),
                                  pl.ds((q - 2) * _W, _W)],
                        osem).wait()
                else:
                    @pl.when(g > 0)
                    def _():
                        pltpu.make_async_copy(
                            h_v,
                            hv_out.at[pl.ds(t0, _G), pl.ds(q * _W, _W)],
                            osem).wait()

                @pl.loop(0, _W // 16)
                def _(dc):
                    koff = q * _W + dc * 16
                    k0 = keys_v[0, pl.ds(koff, 16)]
                    k1 = keys_v[1, pl.ds(koff, 16)]
                    k2 = keys_v[2, pl.ds(koff, 16)]
                    for r in range(_G):
                        h_v[r, pl.ds(dc * 16, 16)] = (
                            k0 * rows_v[3 * r, pl.ds(dc * 16, 16)]
                            + k1 * rows_v[3 * r + 1, pl.ds(dc * 16, 16)]
                            + k2 * rows_v[3 * r + 2, pl.ds(dc * 16, 16)])

                pltpu.async_copy(
                    h_v,
                    hv_out.at[pl.ds(t0 + g * _G, _G), pl.ds(q * _W, _W)],
                    osem)

        pltpu.make_async_copy(
            h0_v, hv_out.at[pl.ds(t0, _G), pl.ds(0, _W)], osem0).wait()
        pltpu.make_async_copy(
            h1_v, hv_out.at[pl.ds(t0, _G), pl.ds(_W, _W)], osem1).wait()

    return gather_k(level2, ftable2, keys_hv, gmain, gfeat)


def _dense_body(h_ref, f_ref, fk_ref, o_ref, carry_ref, acc_ref):
    s = pl.program_id(0)

    @pl.when(s == 0)
    def _():
        carry_ref[...] = jnp.zeros_like(carry_ref)
        acc_ref[...] = jnp.zeros_like(acc_ref)

    h = h_ref[...]
    hcat = jnp.concatenate([carry_ref[...], h], axis=0)
    u = hcat[0:_TC_CHUNK]
    v = hcat[1:_TC_CHUNK + 1]
    w = hcat[2:_TC_CHUNK + 2]
    ur = jnp.concatenate([u[:, -2:], u[:, :-2]], axis=1)
    vr = jnp.concatenate([v[:, -1:], v[:, :-1]], axis=1)
    term = ur * vr * w
    part = acc_ref[...]
    for c in range(_TC_CHUNK // 8):
        part = part + term[c * 8:(c + 1) * 8]
    acc_ref[...] = part
    carry_ref[...] = h[_TC_CHUNK - 2:_TC_CHUNK]

    @pl.when(s == _TC_STEPS - 1)
    def _():
        fhv = jnp.sum(f_ref[...] * fk_ref[...], axis=0, keepdims=True)
        shv = jnp.sum(acc_ref[...], axis=0, keepdims=True)
        comb = shv + fhv + shv * fhv
        o_ref[...] = jnp.where(comb > 0, 1.0, -1.0)


def _dense(hv, feat_rows, feat_keys_pad):
    return pl.pallas_call(
        _dense_body,
        grid=(_TC_STEPS,),
        in_specs=[
            pl.BlockSpec((_TC_CHUNK, _D), lambda s: (s, 0)),
            pl.BlockSpec((24, _D), lambda s: (0, 0)),
            pl.BlockSpec((24, _D), lambda s: (0, 0)),
        ],
        out_specs=pl.BlockSpec((1, _D), lambda s: (0, 0)),
        out_shape=jax.ShapeDtypeStruct((1, _D), jnp.float32),
        scratch_shapes=[
            pltpu.VMEM((2, _D), jnp.float32),
            pltpu.VMEM((8, _D), jnp.float32),
        ],
    )(hv, feat_rows, feat_keys_pad)


@jax.jit
def _run(signals, feat, keys_hv, level_hvs, feat_keys, feat_level_hvs):
    f18 = feat[_FEAT_SEL]
    fpad = jnp.concatenate([f18, jnp.zeros((14,), jnp.float32)]).reshape(8, 4)
    x = jnp.concatenate([signals, fpad], axis=0)
    idx = _quantize(x)
    gmain = idx[:_T, 1:4].reshape(_T * 3)
    gfeat = jnp.zeros((24, 8), jnp.int32).at[:, 0].set(
        idx[_T:_T + 6].reshape(24))
    level2 = _retile(level_hvs)
    hv, feat_rows = _sc_bind_gather(level2, feat_level_hvs, keys_hv,
                                    gmain, gfeat)
    fk_pad = jnp.concatenate(
        [feat_keys, jnp.zeros((6, _D), jnp.float32)], axis=0)
    out = _dense(hv, feat_rows, fk_pad)
    return out.reshape(_D)


def kernel(signals, feat, keys_hv, level_hvs, feat_keys, feat_level_hvs):
    return _run(signals, feat, keys_hv, level_hvs, feat_keys, feat_level_hvs)
